# baseline (speedup 1.0000x reference)
import functools

import jax
import jax.numpy as jnp
from jax import lax
from jax.experimental import pallas as pl
from jax.experimental.pallas import tpu as pltpu
from jax.experimental.pallas import tpu_sc as plsc

ROWS, COLS = 2048, 4096
NUM_ZERO = 2048
LANES = 16
NCHUNK = COLS // LANES
NWORKERS = 32
ROWS_PER_WORKER = ROWS // NWORKERS
NBUCKET = 256
NHIST = 8
UNROLL = 8
SIG_SCALE = 0.8 / (2.0 / 3.0)
TC_BLOCK_ROWS = 128


def _keys_from_bits(b):
    return b ^ ((b >> 31) & 0x7FFFFFFF)


def _sc_find_bucket(hists, kt):
    def scan_chunk(c, carry):
        found, sel, below, cum = carry
        h = hists[0][pl.ds(c * LANES, LANES)]
        for hv in hists[1:]:
            h = h + hv[pl.ds(c * LANES, LANES)]
        cs = plsc.cumsum(h)
        need = kt - cum
        prefix = (cs < need).astype(jnp.int32)
        ff = jnp.sum(prefix)
        below_in = jnp.sum(jnp.where(cs < need, h, 0))
        hit = (1 - found) * jnp.where(ff < LANES, 1, 0)
        sel = jnp.where(hit == 1, c * LANES + ff, sel)
        below = jnp.where(hit == 1, cum + below_in, below)
        found = jnp.where(ff < LANES, 1, found)
        cum = cum + jnp.sum(h)
        return found, sel, below, cum

    init = (jnp.int32(0), jnp.int32(0), jnp.int32(0), jnp.int32(0))
    _, sel, below, _ = lax.fori_loop(0, NBUCKET // LANES, scan_chunk, init)
    return sel, below


def _sc_body(z_hbm, thr_hbm, row_v, key_v, out_v, *hists):
    cid = lax.axis_index("c")
    sid = lax.axis_index("s")
    wid = sid * 2 + cid
    base_row = wid * ROWS_PER_WORKER
    ones = jnp.ones((LANES,), jnp.int32)
    zeros = jnp.zeros((LANES,), jnp.int32)
    lane_iota = lax.iota(jnp.int32, LANES)

    def zero_hist():
        for hv in hists:
            for c in range(NBUCKET // LANES):
                hv[pl.ds(c * LANES, LANES)] = zeros

    def row_threshold(row):
        pltpu.sync_copy(z_hbm.at[row], row_v)

        zero_hist()

        def pass1(jo, _):
            for ji in range(UNROLL):
                j = jo * UNROLL + ji
                z = row_v[pl.ds(j * LANES, LANES)]
                b = lax.bitcast_convert_type(z, jnp.int32)
                k = _keys_from_bits(b)
                key_v[pl.ds(j * LANES, LANES)] = k
                bk = (k >> 24) + 128
                plsc.addupdate_scatter(hists[ji % NHIST], [bk], ones)
            return 0

        lax.fori_loop(0, NCHUNK // UNROLL, pass1, 0)
        sel, below = _sc_find_bucket(hists, jnp.int32(NUM_ZERO))
        prefix = sel - 128
        kt = jnp.int32(NUM_ZERO) - below

        def refine(shift_hi, shift_lo, prefix, kt):
            zero_hist()

            def body(jo, _):
                for ji in range(UNROLL):
                    j = jo * UNROLL + ji
                    k = key_v[pl.ds(j * LANES, LANES)]
                    pm = (k >> shift_hi) == prefix
                    bk = (k >> shift_lo) & 255
                    plsc.addupdate_scatter(
                        hists[ji % NHIST], [bk], ones, mask=pm)
                return 0

            lax.fori_loop(0, NCHUNK // UNROLL, body, 0)
            sel, below = _sc_find_bucket(hists, kt)
            return (prefix << 8) | sel, kt - below

        prefix, kt = refine(24, 16, prefix, kt)
        prefix, kt = refine(16, 8, prefix, kt)
        prefix, kt = refine(8, 0, prefix, kt)
        return prefix

    def group_body(g, _):
        def row_body(i, acc):
            kth = row_threshold(base_row + g * LANES + i)
            return jnp.where(lane_iota == i, kth, acc)

        acc = lax.fori_loop(0, LANES, row_body, zeros)
        out_v[pl.ds(g * LANES, LANES)] = acc
        return 0

    lax.fori_loop(0, ROWS_PER_WORKER // LANES, group_body, 0)
    pltpu.sync_copy(out_v, thr_hbm.at[pl.ds(base_row, ROWS_PER_WORKER)])


_sc_thresholds = functools.partial(
    pl.kernel,
    mesh=plsc.VectorSubcoreMesh(core_axis_name="c", subcore_axis_name="s"),
    out_type=jax.ShapeDtypeStruct((ROWS,), jnp.int32),
    compiler_params=pltpu.CompilerParams(needs_layout_passes=False),
    scratch_types=(
        [
            pltpu.VMEM((COLS,), jnp.float32),
            pltpu.VMEM((COLS,), jnp.int32),
            pltpu.VMEM((ROWS_PER_WORKER,), jnp.int32),
        ]
        + [pltpu.VMEM((NBUCKET,), jnp.int32) for _ in range(NHIST)]
    ),
)(_sc_body)


def _tc_body(z_ref, thr_ref, o_ref):
    z = z_ref[...]
    b = lax.bitcast_convert_type(z, jnp.int32)
    k = _keys_from_bits(b)
    kth = thr_ref[...]
    sig = jax.nn.sigmoid(z * jnp.float32(SIG_SCALE))
    o_ref[...] = jnp.where(k <= kth, jnp.float32(0.0), sig)


@jax.jit
def _impl(z):
    thr = _sc_thresholds(z)
    out = pl.pallas_call(
        _tc_body,
        grid=(ROWS // TC_BLOCK_ROWS,),
        in_specs=[
            pl.BlockSpec((TC_BLOCK_ROWS, COLS), lambda i: (i, 0)),
            pl.BlockSpec((TC_BLOCK_ROWS, 1), lambda i: (i, 0)),
        ],
        out_specs=pl.BlockSpec((TC_BLOCK_ROWS, COLS), lambda i: (i, 0)),
        out_shape=jax.ShapeDtypeStruct((ROWS, COLS), jnp.float32),
    )(z, thr.reshape(ROWS, 1))
    return out


def kernel(z_loga):
    return _impl(z_loga)


# speedup vs baseline: 1.3497x; 1.3497x over previous
import functools

import jax
import jax.numpy as jnp
from jax import lax
from jax.experimental import pallas as pl
from jax.experimental.pallas import tpu as pltpu
from jax.experimental.pallas import tpu_sc as plsc

ROWS, COLS = 2048, 4096
NUM_ZERO = 2048
LANES = 16
NCHUNK = COLS // LANES
NWORKERS = 32
ROWS_PER_WORKER = ROWS // NWORKERS
NBUCKET = 256
UNROLL = 8
SIG_SCALE = 0.8 / (2.0 / 3.0)
TC_BLOCK_ROWS = 128


def _keys_from_bits(b):
    return b ^ ((b >> 31) & 0x7FFFFFFF)


def _sc_find_bucket(hist_v, kt):
    def scan_chunk(c, carry):
        found, sel, below, cum = carry
        h = hist_v[pl.ds(c * LANES, LANES)]
        cs = plsc.cumsum(h)
        need = kt - cum
        prefix = (cs < need).astype(jnp.int32)
        ff = jnp.sum(prefix)
        below_in = jnp.sum(jnp.where(cs < need, h, 0))
        hit = (1 - found) * jnp.where(ff < LANES, 1, 0)
        sel = jnp.where(hit == 1, c * LANES + ff, sel)
        below = jnp.where(hit == 1, cum + below_in, below)
        found = jnp.where(ff < LANES, 1, found)
        cum = cum + jnp.sum(h)
        return found, sel, below, cum

    init = (jnp.int32(0), jnp.int32(0), jnp.int32(0), jnp.int32(0))
    _, sel, below, _ = lax.fori_loop(0, NBUCKET // LANES, scan_chunk, init)
    return sel, below


def _sc_body(z_hbm, thr_hbm, row_v, key_v, cand_v, out_v, hist_v):
    cid = lax.axis_index("c")
    sid = lax.axis_index("s")
    wid = sid * 2 + cid
    base_row = wid * ROWS_PER_WORKER
    ones = jnp.ones((LANES,), jnp.int32)
    zeros = jnp.zeros((LANES,), jnp.int32)
    lane_iota = lax.iota(jnp.int32, LANES)

    def zero_hist():
        for c in range(NBUCKET // LANES):
            hist_v[pl.ds(c * LANES, LANES)] = zeros

    def row_threshold(row):
        pltpu.sync_copy(z_hbm.at[row], row_v)

        zero_hist()

        def pass1(jo, _):
            for ji in range(UNROLL):
                j = jo * UNROLL + ji
                z = row_v[pl.ds(j * LANES, LANES)]
                b = lax.bitcast_convert_type(z, jnp.int32)
                k = _keys_from_bits(b)
                key_v[pl.ds(j * LANES, LANES)] = k
                bk = (k >> 24) + 128
                plsc.addupdate_scatter(hist_v, [bk], ones)
            return 0

        lax.fori_loop(0, NCHUNK // UNROLL, pass1, 0)
        sel, below = _sc_find_bucket(hist_v, jnp.int32(NUM_ZERO))
        prefix = sel - 128
        kt = jnp.int32(NUM_ZERO) - below

        def compact(jo, ptr):
            for ji in range(UNROLL):
                j = jo * UNROLL + ji
                k = key_v[pl.ds(j * LANES, LANES)]
                pm = (k >> 24) == prefix
                plsc.store_compressed(
                    cand_v.at[pl.ds(ptr, LANES)], k, mask=pm)
                ptr = ptr + jnp.sum(pm.astype(jnp.int32))
            return ptr

        ncand = lax.fori_loop(0, NCHUNK // UNROLL, compact, jnp.int32(0))
        nch = (ncand + (LANES - 1)) >> 4

        def refine(shift_hi, shift_lo, prefix, kt, check_prefix):
            zero_hist()

            def body(j, _):
                k = cand_v[pl.ds(j * LANES, LANES)]
                pm = lane_iota < (ncand - j * LANES)
                if check_prefix:
                    pm = jnp.logical_and(pm, (k >> shift_hi) == prefix)
                bk = (k >> shift_lo) & 255
                plsc.addupdate_scatter(hist_v, [bk], ones, mask=pm)
                return 0

            lax.fori_loop(0, nch, body, 0)
            sel, below = _sc_find_bucket(hist_v, kt)
            return (prefix << 8) | sel, kt - below

        prefix, kt = refine(24, 16, prefix, kt, False)
        prefix, kt = refine(16, 8, prefix, kt, True)
        prefix, kt = refine(8, 0, prefix, kt, True)
        return prefix

    def group_body(g, _):
        def row_body(i, acc):
            kth = row_threshold(base_row + g * LANES + i)
            return jnp.where(lane_iota == i, kth, acc)

        acc = lax.fori_loop(0, LANES, row_body, zeros)
        out_v[pl.ds(g * LANES, LANES)] = acc
        return 0

    lax.fori_loop(0, ROWS_PER_WORKER // LANES, group_body, 0)
    pltpu.sync_copy(out_v, thr_hbm.at[pl.ds(base_row, ROWS_PER_WORKER)])


_sc_thresholds = functools.partial(
    pl.kernel,
    mesh=plsc.VectorSubcoreMesh(core_axis_name="c", subcore_axis_name="s"),
    out_type=jax.ShapeDtypeStruct((ROWS,), jnp.int32),
    compiler_params=pltpu.CompilerParams(needs_layout_passes=False),
    scratch_types=[
        pltpu.VMEM((COLS,), jnp.float32),
        pltpu.VMEM((COLS,), jnp.int32),
        pltpu.VMEM((COLS + LANES,), jnp.int32),
        pltpu.VMEM((ROWS_PER_WORKER,), jnp.int32),
        pltpu.VMEM((NBUCKET,), jnp.int32),
    ],
)(_sc_body)


def _tc_body(z_ref, thr_ref, o_ref):
    z = z_ref[...]
    b = lax.bitcast_convert_type(z, jnp.int32)
    k = _keys_from_bits(b)
    kth = thr_ref[...]
    sig = jax.nn.sigmoid(z * jnp.float32(SIG_SCALE))
    o_ref[...] = jnp.where(k <= kth, jnp.float32(0.0), sig)


@jax.jit
def _impl(z):
    thr = _sc_thresholds(z)
    out = pl.pallas_call(
        _tc_body,
        grid=(ROWS // TC_BLOCK_ROWS,),
        in_specs=[
            pl.BlockSpec((TC_BLOCK_ROWS, COLS), lambda i: (i, 0)),
            pl.BlockSpec((TC_BLOCK_ROWS, 1), lambda i: (i, 0)),
        ],
        out_specs=pl.BlockSpec((TC_BLOCK_ROWS, COLS), lambda i: (i, 0)),
        out_shape=jax.ShapeDtypeStruct((ROWS, COLS), jnp.float32),
    )(z, thr.reshape(ROWS, 1))
    return out


def kernel(z_loga):
    return _impl(z_loga)


# speedup vs baseline: 1.4396x; 1.0667x over previous
import functools

import jax
import jax.numpy as jnp
from jax import lax
from jax.experimental import pallas as pl
from jax.experimental.pallas import tpu as pltpu
from jax.experimental.pallas import tpu_sc as plsc

ROWS, COLS = 2048, 4096
NUM_ZERO = 2048
LANES = 16
NCHUNK = COLS // LANES
NWORKERS = 32
ROWS_PER_WORKER = ROWS // NWORKERS
NBUCKET = 256
UNROLL = 8
SIG_SCALE = 0.8 / (2.0 / 3.0)
TC_BLOCK_ROWS = 128


def _keys_from_bits(b):
    return b ^ ((b >> 31) & 0x7FFFFFFF)


def _sc_find_bucket(hist_v, kt):
    def scan_chunk(c, carry):
        found, sel, below, cum = carry
        h = hist_v[pl.ds(c * LANES, LANES)]
        cs = plsc.cumsum(h)
        need = kt - cum
        prefix = (cs < need).astype(jnp.int32)
        ff = jnp.sum(prefix)
        below_in = jnp.sum(jnp.where(cs < need, h, 0))
        hit = (1 - found) * jnp.where(ff < LANES, 1, 0)
        sel = jnp.where(hit == 1, c * LANES + ff, sel)
        below = jnp.where(hit == 1, cum + below_in, below)
        found = jnp.where(ff < LANES, 1, found)
        cum = cum + jnp.sum(h)
        return found, sel, below, cum

    init = (jnp.int32(0), jnp.int32(0), jnp.int32(0), jnp.int32(0))
    _, sel, below, _ = lax.fori_loop(0, NBUCKET // LANES, scan_chunk, init)
    return sel, below


def _sc_body(z_hbm, thr_hbm, row_v, key_v, cand_v, out_v, hist_v):
    cid = lax.axis_index("c")
    sid = lax.axis_index("s")
    wid = sid * 2 + cid
    base_row = wid * ROWS_PER_WORKER
    ones = jnp.ones((LANES,), jnp.int32)
    zeros = jnp.zeros((LANES,), jnp.int32)
    lane_iota = lax.iota(jnp.int32, LANES)

    def zero_hist():
        for c in range(NBUCKET // LANES):
            hist_v[pl.ds(c * LANES, LANES)] = zeros

    def row_threshold(row):
        pltpu.sync_copy(z_hbm.at[row], row_v)

        zero_hist()

        def pass1(jo, _):
            for ji in range(UNROLL):
                j = jo * UNROLL + ji
                z = row_v[pl.ds(j * LANES, LANES)]
                b = lax.bitcast_convert_type(z, jnp.int32)
                k = _keys_from_bits(b)
                key_v[pl.ds(j * LANES, LANES)] = k
                bk = (k >> 24) + 128
                plsc.addupdate_scatter(hist_v, [bk], ones)
            return 0

        lax.fori_loop(0, NCHUNK // UNROLL, pass1, 0)
        sel, below = _sc_find_bucket(hist_v, jnp.int32(NUM_ZERO))
        prefix = sel - 128
        kt = jnp.int32(NUM_ZERO) - below

        def compact(jo, ptr):
            for ji in range(UNROLL):
                j = jo * UNROLL + ji
                k = key_v[pl.ds(j * LANES, LANES)]
                pm = (k >> 24) == prefix
                plsc.store_compressed(
                    cand_v.at[pl.ds(ptr, LANES)], k, mask=pm)
                pc = plsc.all_reduce_population_count(pm)
                ptr = ptr + pc[0]
            return ptr

        ncand = lax.fori_loop(0, NCHUNK // UNROLL, compact, jnp.int32(0))
        nch = (ncand + (LANES - 1)) >> 4

        def refine(shift_hi, shift_lo, prefix, kt, check_prefix):
            zero_hist()

            def body(j, _):
                k = cand_v[pl.ds(j * LANES, LANES)]
                pm = lane_iota < (ncand - j * LANES)
                if check_prefix:
                    pm = jnp.logical_and(pm, (k >> shift_hi) == prefix)
                bk = (k >> shift_lo) & 255
                plsc.addupdate_scatter(hist_v, [bk], ones, mask=pm)
                return 0

            lax.fori_loop(0, nch, body, 0)
            sel, below = _sc_find_bucket(hist_v, kt)
            return (prefix << 8) | sel, kt - below

        prefix, kt = refine(24, 16, prefix, kt, False)
        prefix, kt = refine(16, 8, prefix, kt, True)
        prefix, kt = refine(8, 0, prefix, kt, True)
        return prefix

    def group_body(g, _):
        def row_body(i, acc):
            kth = row_threshold(base_row + g * LANES + i)
            return jnp.where(lane_iota == i, kth, acc)

        acc = lax.fori_loop(0, LANES, row_body, zeros)
        out_v[pl.ds(g * LANES, LANES)] = acc
        return 0

    lax.fori_loop(0, ROWS_PER_WORKER // LANES, group_body, 0)
    pltpu.sync_copy(out_v, thr_hbm.at[pl.ds(base_row, ROWS_PER_WORKER)])


_sc_thresholds = functools.partial(
    pl.kernel,
    mesh=plsc.VectorSubcoreMesh(core_axis_name="c", subcore_axis_name="s"),
    out_type=jax.ShapeDtypeStruct((ROWS,), jnp.int32),
    compiler_params=pltpu.CompilerParams(needs_layout_passes=False),
    scratch_types=[
        pltpu.VMEM((COLS,), jnp.float32),
        pltpu.VMEM((COLS,), jnp.int32),
        pltpu.VMEM((COLS + LANES,), jnp.int32),
        pltpu.VMEM((ROWS_PER_WORKER,), jnp.int32),
        pltpu.VMEM((NBUCKET,), jnp.int32),
    ],
)(_sc_body)


def _tc_body(z_ref, thr_ref, o_ref):
    z = z_ref[...]
    b = lax.bitcast_convert_type(z, jnp.int32)
    k = _keys_from_bits(b)
    kth = thr_ref[...]
    sig = jax.nn.sigmoid(z * jnp.float32(SIG_SCALE))
    o_ref[...] = jnp.where(k <= kth, jnp.float32(0.0), sig)


@jax.jit
def _impl(z):
    thr = _sc_thresholds(z)
    out = pl.pallas_call(
        _tc_body,
        grid=(ROWS // TC_BLOCK_ROWS,),
        in_specs=[
            pl.BlockSpec((TC_BLOCK_ROWS, COLS), lambda i: (i, 0)),
            pl.BlockSpec((TC_BLOCK_ROWS, 1), lambda i: (i, 0)),
        ],
        out_specs=pl.BlockSpec((TC_BLOCK_ROWS, COLS), lambda i: (i, 0)),
        out_shape=jax.ShapeDtypeStruct((ROWS, COLS), jnp.float32),
    )(z, thr.reshape(ROWS, 1))
    return out


def kernel(z_loga):
    return _impl(z_loga)
